# baseline (speedup 1.0000x reference)
import functools

import jax
import jax.numpy as jnp
from jax import lax
from jax.experimental import pallas as pl
from jax.experimental.pallas import tpu as pltpu
from jax.experimental.pallas import tpu_sc as plsc

_G = 64
_T = 12
_K = _G * _G
_NW = 32
_ROWS_W = _K // _NW
_CHUNK = 8
_NCHUNK = _ROWS_W // _CHUNK
_STRIPE = 128
_NSTRIPE = _K // _STRIPE
_VPS = _STRIPE // 16


def _sc_partial_body(od_hbm, mu0_hbm, out_hbm,
                     buf0, buf1, acc_b, acc_c, mu_v, sem0, sem1):
    wid = lax.axis_index("s") * 2 + lax.axis_index("c")
    row0 = wid * _ROWS_W

    pltpu.sync_copy(mu0_hbm.at[pl.ds(row0, _ROWS_W)],
                    mu_v.at[pl.ds(0, _ROWS_W)])

    zeros16 = jnp.zeros((16,), jnp.float32)

    def _zero(i, carry):
        acc_b[pl.ds(i * 16, 16)] = zeros16
        acc_c[pl.ds(i * 16, 16)] = zeros16
        return carry

    lax.fori_loop(0, _K // 16, _zero, 0)

    bufs = (buf0, buf1)
    sems = (sem0, sem1)

    def _start(c):
        return pltpu.async_copy(
            od_hbm.at[pl.ds((row0 + c * _CHUNK) * _K, _CHUNK * _K)],
            bufs[c % 2], sems[c % 2])

    handles = {0: _start(0)}
    for c in range(_NCHUNK):
        if c + 1 < _NCHUNK:
            handles[c + 1] = _start(c + 1)
        handles[c].wait()
        buf = bufs[c % 2]

        def _stripe(s, carry, c=c, buf=buf):
            col0 = s * _STRIPE
            accs_b = tuple(acc_b[pl.ds(col0 + j * 16, 16)]
                           for j in range(_VPS))
            accs_c = tuple(acc_c[pl.ds(col0 + j * 16, 16)]
                           for j in range(_VPS))

            def _row(r, rc):
                ab, ac = rc
                mu = mu_v[pl.ds(c * _CHUNK + r, 16)][0]
                base = r * _K + col0
                ods = tuple(buf[pl.ds(base + j * 16, 16)]
                            for j in range(_VPS))
                ab = tuple(ab[j] + mu * ods[j] for j in range(_VPS))
                ac = tuple(ac[j] + ods[j] for j in range(_VPS))
                return (ab, ac)

            ab, ac = lax.fori_loop(0, _CHUNK, _row, (accs_b, accs_c))
            for j in range(_VPS):
                acc_b[pl.ds(col0 + j * 16, 16)] = ab[j]
                acc_c[pl.ds(col0 + j * 16, 16)] = ac[j]
            return carry

        lax.fori_loop(0, _NSTRIPE, _stripe, 0)

    out_base = wid * _K
    pltpu.sync_copy(acc_b, out_hbm.at[pl.ds(out_base, _K)])
    pltpu.sync_copy(acc_c, out_hbm.at[pl.ds(_NW * _K + out_base, _K)])


_sc_partial = pl.kernel(
    _sc_partial_body,
    out_type=jax.ShapeDtypeStruct((2 * _NW * _K,), jnp.float32),
    mesh=plsc.VectorSubcoreMesh(core_axis_name="c", subcore_axis_name="s"),
    scratch_types=[
        pltpu.VMEM((_CHUNK * _K,), jnp.float32),
        pltpu.VMEM((_CHUNK * _K,), jnp.float32),
        pltpu.VMEM((_K,), jnp.float32),
        pltpu.VMEM((_K,), jnp.float32),
        pltpu.VMEM((_ROWS_W + 16,), jnp.float32),
        pltpu.SemaphoreType.DMA,
        pltpu.SemaphoreType.DMA,
    ],
)


def _combine_kernel(parts_ref, mu0_ref, harm_ref, out_ref):
    base = jnp.sum(parts_ref[0:_NW, :], axis=0, keepdims=True)
    colsum = jnp.sum(parts_ref[_NW:2 * _NW, :], axis=0, keepdims=True)
    harm = harm_ref[:, 0:1]
    mu0_flat = mu0_ref[...]
    out_ref[:, 0, :] = mu0_flat + harm
    out_ref[:, 1, :] = base + harm * colsum


def kernel(t_array, mu0, a_k, b_k, od_matrix):
    od_flat = od_matrix.reshape(_K * _K)
    mu0_flat = mu0.reshape(_K).astype(jnp.float32)

    t_norm = 2.0 * jnp.pi * t_array / 120.0
    harm = (a_k[0] * jnp.sin(t_norm) + b_k[0] * jnp.cos(t_norm)
            + a_k[1] * jnp.sin(2.0 * t_norm) + b_k[1] * jnp.cos(2.0 * t_norm))
    harm2 = jnp.broadcast_to(harm[:, None], (_T, 128)).astype(jnp.float32)

    parts = _sc_partial(od_flat, mu0_flat).reshape(2 * _NW, _K)

    out = pl.pallas_call(
        _combine_kernel,
        in_specs=[
            pl.BlockSpec((2 * _NW, _K), lambda: (0, 0)),
            pl.BlockSpec((1, _K), lambda: (0, 0)),
            pl.BlockSpec((_T, 128), lambda: (0, 0)),
        ],
        out_specs=pl.BlockSpec((_T, 2, _K), lambda: (0, 0, 0)),
        out_shape=jax.ShapeDtypeStruct((_T, 2, _K), jnp.float32),
    )(parts, mu0_flat.reshape(1, _K), harm2)

    return out.reshape(_T, 2, _G, _G)


# speedup vs baseline: 3.2186x; 3.2186x over previous
import jax
import jax.numpy as jnp
from jax import lax
from jax.experimental import pallas as pl
from jax.experimental.pallas import tpu as pltpu

_G = 64
_T = 12
_BI = 4
_NBLK = _G // _BI


def _reduce_kernel(od_ref, w_ref, mu0_ref, harm_ref, out_ref, acc_b, acc_c):
    k = pl.program_id(0)

    @pl.when(k == 0)
    def _init():
        acc_b[...] = jnp.zeros_like(acc_b)
        acc_c[...] = jnp.zeros_like(acc_c)

    def _col(j, accs):
        ab, ac = accs
        for i in range(_BI):
            slab = od_ref[i, j]
            w = w_ref[0, (k * _BI + i) * _G + j]
            ab = ab + w * slab
            ac = ac + slab
        return (ab, ac)

    ab, ac = lax.fori_loop(0, _G, _col, (acc_b[...], acc_c[...]))
    acc_b[...] = ab
    acc_c[...] = ac

    @pl.when(k == _NBLK - 1)
    def _finish():
        harm = harm_ref[:, 0].reshape(_T, 1, 1)
        mu0 = mu0_ref[...]
        out_ref[:, 0, :, :] = mu0[None, :, :] + harm
        out_ref[:, 1, :, :] = ab[None, :, :] + harm * ac[None, :, :]


def kernel(t_array, mu0, a_k, b_k, od_matrix):
    mu0 = mu0.astype(jnp.float32)
    w_flat = mu0.reshape(1, _G * _G)

    t_norm = 2.0 * jnp.pi * t_array / 120.0
    harm = (a_k[0] * jnp.sin(t_norm) + b_k[0] * jnp.cos(t_norm)
            + a_k[1] * jnp.sin(2.0 * t_norm) + b_k[1] * jnp.cos(2.0 * t_norm))
    harm2 = jnp.broadcast_to(harm[:, None], (_T, 128)).astype(jnp.float32)

    out = pl.pallas_call(
        _reduce_kernel,
        grid=(_NBLK,),
        in_specs=[
            pl.BlockSpec((_BI, _G, _G, _G), lambda k: (k, 0, 0, 0)),
            pl.BlockSpec((1, _G * _G), lambda k: (0, 0),
                         memory_space=pltpu.SMEM),
            pl.BlockSpec((_G, _G), lambda k: (0, 0)),
            pl.BlockSpec((_T, 128), lambda k: (0, 0)),
        ],
        out_specs=pl.BlockSpec((_T, 2, _G, _G), lambda k: (0, 0, 0, 0)),
        out_shape=jax.ShapeDtypeStruct((_T, 2, _G, _G), jnp.float32),
        scratch_shapes=[
            pltpu.VMEM((_G, _G), jnp.float32),
            pltpu.VMEM((_G, _G), jnp.float32),
        ],
        compiler_params=pltpu.CompilerParams(
            dimension_semantics=("arbitrary",)),
    )(od_matrix, w_flat, mu0, harm2)

    return out


# speedup vs baseline: 3.2577x; 1.0122x over previous
import jax
import jax.numpy as jnp
from jax import lax
from jax.experimental import pallas as pl
from jax.experimental.pallas import tpu as pltpu

_G = 64
_T = 12
_BI = 8
_NBLK = _G // _BI


def _reduce_kernel(od_ref, w_ref, mu0_ref, harm_ref, out_ref, acc_b, acc_c):
    k = pl.program_id(0)

    @pl.when(k == 0)
    def _init():
        acc_b[...] = jnp.zeros_like(acc_b)
        acc_c[...] = jnp.zeros_like(acc_c)

    def _col(j, accs):
        ab, ac = accs
        for i in range(_BI):
            slab = od_ref[i, j]
            w = w_ref[0, (k * _BI + i) * _G + j]
            ab = ab + w * slab
            ac = ac + slab
        return (ab, ac)

    ab, ac = lax.fori_loop(0, _G, _col, (acc_b[...], acc_c[...]))
    acc_b[...] = ab
    acc_c[...] = ac

    @pl.when(k == _NBLK - 1)
    def _finish():
        harm = harm_ref[:, 0].reshape(_T, 1, 1)
        mu0 = mu0_ref[...]
        out_ref[:, 0, :, :] = mu0[None, :, :] + harm
        out_ref[:, 1, :, :] = ab[None, :, :] + harm * ac[None, :, :]


def kernel(t_array, mu0, a_k, b_k, od_matrix):
    mu0 = mu0.astype(jnp.float32)
    w_flat = mu0.reshape(1, _G * _G)

    t_norm = 2.0 * jnp.pi * t_array / 120.0
    harm = (a_k[0] * jnp.sin(t_norm) + b_k[0] * jnp.cos(t_norm)
            + a_k[1] * jnp.sin(2.0 * t_norm) + b_k[1] * jnp.cos(2.0 * t_norm))
    harm2 = jnp.broadcast_to(harm[:, None], (_T, 128)).astype(jnp.float32)

    out = pl.pallas_call(
        _reduce_kernel,
        grid=(_NBLK,),
        in_specs=[
            pl.BlockSpec((_BI, _G, _G, _G), lambda k: (k, 0, 0, 0)),
            pl.BlockSpec((1, _G * _G), lambda k: (0, 0),
                         memory_space=pltpu.SMEM),
            pl.BlockSpec((_G, _G), lambda k: (0, 0)),
            pl.BlockSpec((_T, 128), lambda k: (0, 0)),
        ],
        out_specs=pl.BlockSpec((_T, 2, _G, _G), lambda k: (0, 0, 0, 0)),
        out_shape=jax.ShapeDtypeStruct((_T, 2, _G, _G), jnp.float32),
        scratch_shapes=[
            pltpu.VMEM((_G, _G), jnp.float32),
            pltpu.VMEM((_G, _G), jnp.float32),
        ],
        compiler_params=pltpu.CompilerParams(
            dimension_semantics=("arbitrary",)),
    )(od_matrix, w_flat, mu0, harm2)

    return out


# speedup vs baseline: 3.7528x; 1.1520x over previous
import jax
import jax.numpy as jnp
from jax import lax
from jax.experimental import pallas as pl
from jax.experimental.pallas import tpu as pltpu

_G = 64
_T = 12
_BI = 8
_NBLK = _G // _BI


def _reduce_kernel(od_ref, w_ref, mu0_ref, t_ref, ak_ref, bk_ref,
                   out_ref, acc_b, acc_c):
    k = pl.program_id(0)

    @pl.when(k == 0)
    def _init():
        acc_b[...] = jnp.zeros_like(acc_b)
        acc_c[...] = jnp.zeros_like(acc_c)

    def _col(j, accs):
        ab, ac = accs
        for i in range(_BI):
            slab = od_ref[i, j]
            w = w_ref[k * _BI + i, j]
            ab = ab + w * slab
            ac = ac + slab
        return (ab, ac)

    ab, ac = lax.fori_loop(0, _G, _col, (acc_b[...], acc_c[...]))
    acc_b[...] = ab
    acc_c[...] = ac

    @pl.when(k == _NBLK - 1)
    def _finish():
        t_norm = t_ref[...] * (2.0 * jnp.pi / 120.0)
        harm = (ak_ref[0] * jnp.sin(t_norm) + bk_ref[0] * jnp.cos(t_norm)
                + ak_ref[1] * jnp.sin(2.0 * t_norm)
                + bk_ref[1] * jnp.cos(2.0 * t_norm))
        harm3 = harm[:, 0].reshape(_T, 1, 1)
        mu0 = mu0_ref[...]
        out_ref[:, 0, :, :] = mu0[None, :, :] + harm3
        out_ref[:, 1, :, :] = ab[None, :, :] + harm3 * ac[None, :, :]


def kernel(t_array, mu0, a_k, b_k, od_matrix):
    t128 = jnp.broadcast_to(t_array[:, None], (_T, 128))

    out = pl.pallas_call(
        _reduce_kernel,
        grid=(_NBLK,),
        in_specs=[
            pl.BlockSpec((_BI, _G, _G, _G), lambda k: (k, 0, 0, 0)),
            pl.BlockSpec((_G, _G), lambda k: (0, 0),
                         memory_space=pltpu.SMEM),
            pl.BlockSpec((_G, _G), lambda k: (0, 0)),
            pl.BlockSpec((_T, 128), lambda k: (0, 0)),
            pl.BlockSpec(memory_space=pltpu.SMEM),
            pl.BlockSpec(memory_space=pltpu.SMEM),
        ],
        out_specs=pl.BlockSpec((_T, 2, _G, _G), lambda k: (0, 0, 0, 0)),
        out_shape=jax.ShapeDtypeStruct((_T, 2, _G, _G), jnp.float32),
        scratch_shapes=[
            pltpu.VMEM((_G, _G), jnp.float32),
            pltpu.VMEM((_G, _G), jnp.float32),
        ],
        compiler_params=pltpu.CompilerParams(
            dimension_semantics=("arbitrary",)),
    )(od_matrix, mu0, mu0, t128, a_k, b_k)

    return out
